# baseline (speedup 1.0000x reference)
import functools

import jax
import jax.numpy as jnp
from jax.experimental import pallas as pl
import jax.experimental.pallas.tpu as pltpu

N = 4096
H1 = 128
H2 = 64
D_IN = 256


def _mm_kernel(x_ref, w_ref, o_ref):
    o_ref[...] = jnp.dot(x_ref[...], w_ref[...],
                         preferred_element_type=jnp.float32)


def _small_matmul(x, w):
    return pl.pallas_call(
        _mm_kernel,
        out_shape=jax.ShapeDtypeStruct((x.shape[0], w.shape[1]), jnp.float32),
    )(x, w)


def _layer1_kernel(adj_ref, t_ref, scale_ref, beta_ref, hs_ref):
    x = jnp.dot(adj_ref[...], t_ref[...], preferred_element_type=jnp.float32)
    hs_ref[...] = jnp.maximum(x, 0.0) * scale_ref[...] + beta_ref[...]


def _layer2_kernel(adj_ref, u_ref, s1r_ref, s1c_ref, sc_ref, b_ref,
                   feat_ref, struct_ref, acc_ref):
    j = pl.program_id(1)

    @pl.when(j == 0)
    def _():
        acc_ref[...] = jnp.zeros_like(acc_ref)

    acc_ref[...] += jnp.dot(adj_ref[...], u_ref[...],
                            preferred_element_type=jnp.float32)
    struct_ref[...] = jax.lax.dot_general(
        s1r_ref[...], s1c_ref[...],
        (((1,), (1,)), ((), ())), preferred_element_type=jnp.float32)

    @pl.when(j == pl.num_programs(1) - 1)
    def _():
        feat_ref[...] = (jnp.maximum(acc_ref[...], 0.0) * sc_ref[...]
                         + b_ref[...])


def kernel(y_features, adj, W_fd1, W_fd2, W_sd1, g1, b1, g2, b2, g3, b3):
    inv = 1.0 / jnp.sqrt(jnp.float32(1.0 + 1e-5))
    w_cat = jnp.concatenate([W_fd1, W_sd1], axis=1)
    scale_cat = (jnp.concatenate([g1, g3]) * inv).reshape(1, 2 * H1)
    beta_cat = jnp.concatenate([b1, b3]).reshape(1, 2 * H1)
    sc2 = (g2 * inv).reshape(1, D_IN)
    b2r = b2.reshape(1, D_IN)

    t = _small_matmul(y_features, w_cat)

    BM1 = 512
    hs = pl.pallas_call(
        _layer1_kernel,
        grid=(N // BM1,),
        in_specs=[
            pl.BlockSpec((BM1, N), lambda i: (i, 0)),
            pl.BlockSpec((N, 2 * H1), lambda i: (0, 0)),
            pl.BlockSpec((1, 2 * H1), lambda i: (0, 0)),
            pl.BlockSpec((1, 2 * H1), lambda i: (0, 0)),
        ],
        out_specs=pl.BlockSpec((BM1, 2 * H1), lambda i: (i, 0)),
        out_shape=jax.ShapeDtypeStruct((N, 2 * H1), jnp.float32),
    )(adj, t, scale_cat, beta_cat)

    h = hs[:, :H1]
    s1 = hs[:, H1:]

    u = _small_matmul(h, W_fd2)

    BM = 512
    BN = 512
    feat, struct = pl.pallas_call(
        _layer2_kernel,
        grid=(N // BM, N // BN),
        in_specs=[
            pl.BlockSpec((BM, BN), lambda i, j: (i, j)),
            pl.BlockSpec((BN, D_IN), lambda i, j: (j, 0)),
            pl.BlockSpec((BM, H1), lambda i, j: (i, 0)),
            pl.BlockSpec((BN, H1), lambda i, j: (j, 0)),
            pl.BlockSpec((1, D_IN), lambda i, j: (0, 0)),
            pl.BlockSpec((1, D_IN), lambda i, j: (0, 0)),
        ],
        out_specs=[
            pl.BlockSpec((BM, D_IN), lambda i, j: (i, 0)),
            pl.BlockSpec((BM, BN), lambda i, j: (i, j)),
        ],
        out_shape=[
            jax.ShapeDtypeStruct((N, D_IN), jnp.float32),
            jax.ShapeDtypeStruct((N, N), jnp.float32),
        ],
        scratch_shapes=[pltpu.VMEM((BM, D_IN), jnp.float32)],
    )(adj, u, s1, s1, sc2, b2r)

    return (feat, struct)


# speedup vs baseline: 1.1762x; 1.1762x over previous
import jax
import jax.numpy as jnp
from jax.experimental import pallas as pl
import jax.experimental.pallas.tpu as pltpu

N = 4096
H1 = 128
H2 = 64
D_IN = 256

BM1 = 512
BM = 512
BN = 512


def _pass1_kernel(adj_ref, y_ref, w_ref, scale_ref, beta_ref,
                  h_ref, s1_ref, t_ref):
    @pl.when(pl.program_id(0) == 0)
    def _():
        t_ref[...] = jnp.dot(y_ref[...], w_ref[...],
                             preferred_element_type=jnp.float32)

    x = jnp.dot(adj_ref[...], t_ref[...], preferred_element_type=jnp.float32)
    hs = jnp.maximum(x, 0.0) * scale_ref[...] + beta_ref[...]
    h_ref[...] = hs[:, :H1]
    s1_ref[...] = hs[:, H1:]


def _pass2_kernel(adj_ref, h_ref, s1_ref, w2_ref, sc_ref, b_ref,
                  feat_ref, struct_ref, u_ref, acc_ref):
    i = pl.program_id(0)
    j = pl.program_id(1)

    @pl.when(jnp.logical_and(i == 0, j == 0))
    def _():
        u_ref[...] = jnp.dot(h_ref[...], w2_ref[...],
                             preferred_element_type=jnp.float32)

    @pl.when(j == 0)
    def _():
        acc_ref[...] = jnp.zeros_like(acc_ref)

    acc_ref[...] += jnp.dot(adj_ref[...], u_ref[pl.ds(j * BN, BN), :],
                            preferred_element_type=jnp.float32)
    struct_ref[...] = jax.lax.dot_general(
        s1_ref[pl.ds(i * BM, BM), :], s1_ref[pl.ds(j * BN, BN), :],
        (((1,), (1,)), ((), ())), preferred_element_type=jnp.float32)

    @pl.when(j == pl.num_programs(1) - 1)
    def _():
        feat_ref[...] = (jnp.maximum(acc_ref[...], 0.0) * sc_ref[...]
                         + b_ref[...])


def kernel(y_features, adj, W_fd1, W_fd2, W_sd1, g1, b1, g2, b2, g3, b3):
    inv = 1.0 / jnp.sqrt(jnp.float32(1.0 + 1e-5))
    w_cat = jnp.concatenate([W_fd1, W_sd1], axis=1)
    scale_cat = (jnp.concatenate([g1, g3]) * inv).reshape(1, 2 * H1)
    beta_cat = jnp.concatenate([b1, b3]).reshape(1, 2 * H1)
    sc2 = (g2 * inv).reshape(1, D_IN)
    b2r = b2.reshape(1, D_IN)

    h, s1 = pl.pallas_call(
        _pass1_kernel,
        grid=(N // BM1,),
        in_specs=[
            pl.BlockSpec((BM1, N), lambda i: (i, 0)),
            pl.BlockSpec((N, H2), lambda i: (0, 0)),
            pl.BlockSpec((H2, 2 * H1), lambda i: (0, 0)),
            pl.BlockSpec((1, 2 * H1), lambda i: (0, 0)),
            pl.BlockSpec((1, 2 * H1), lambda i: (0, 0)),
        ],
        out_specs=[
            pl.BlockSpec((BM1, H1), lambda i: (i, 0)),
            pl.BlockSpec((BM1, H1), lambda i: (i, 0)),
        ],
        out_shape=[
            jax.ShapeDtypeStruct((N, H1), jnp.float32),
            jax.ShapeDtypeStruct((N, H1), jnp.float32),
        ],
        scratch_shapes=[pltpu.VMEM((N, 2 * H1), jnp.float32)],
    )(adj, y_features, w_cat, scale_cat, beta_cat)

    feat, struct = pl.pallas_call(
        _pass2_kernel,
        grid=(N // BM, N // BN),
        in_specs=[
            pl.BlockSpec((BM, BN), lambda i, j: (i, j)),
            pl.BlockSpec((N, H1), lambda i, j: (0, 0)),
            pl.BlockSpec((N, H1), lambda i, j: (0, 0)),
            pl.BlockSpec((H1, D_IN), lambda i, j: (0, 0)),
            pl.BlockSpec((1, D_IN), lambda i, j: (0, 0)),
            pl.BlockSpec((1, D_IN), lambda i, j: (0, 0)),
        ],
        out_specs=[
            pl.BlockSpec((BM, D_IN), lambda i, j: (i, 0)),
            pl.BlockSpec((BM, BN), lambda i, j: (i, j)),
        ],
        out_shape=[
            jax.ShapeDtypeStruct((N, D_IN), jnp.float32),
            jax.ShapeDtypeStruct((N, N), jnp.float32),
        ],
        scratch_shapes=[
            pltpu.VMEM((N, D_IN), jnp.float32),
            pltpu.VMEM((BM, D_IN), jnp.float32),
        ],
    )(adj, h, s1, W_fd2, sc2, b2r)

    return (feat, struct)


# speedup vs baseline: 1.5757x; 1.3397x over previous
import jax
import jax.numpy as jnp
from jax.experimental import pallas as pl
import jax.experimental.pallas.tpu as pltpu

N = 4096
H1 = 128
H2 = 64
D_IN = 256

BM1 = 1024
BM = 1024
BN = 1024


def _pass1_kernel(adj_ref, y_ref, w_ref, scale_ref, beta_ref,
                  h_ref, s1_ref, t_ref):
    @pl.when(pl.program_id(0) == 0)
    def _():
        t_ref[...] = jnp.dot(y_ref[...], w_ref[...],
                             preferred_element_type=jnp.float32)

    x = jnp.dot(adj_ref[...], t_ref[...], preferred_element_type=jnp.float32)
    hs = jnp.maximum(x, 0.0) * scale_ref[...] + beta_ref[...]
    h_ref[...] = hs[:, :H1]
    s1_ref[...] = hs[:, H1:]


def _pass2_kernel(adj_ref, h_ref, s1_ref, w2_ref, sc_ref, b_ref,
                  feat_ref, struct_ref, u_ref, acc_ref):
    i = pl.program_id(0)
    j = pl.program_id(1)

    @pl.when(jnp.logical_and(i == 0, j == 0))
    def _():
        u_ref[...] = jnp.dot(h_ref[...], w2_ref[...],
                             preferred_element_type=jnp.float32)

    @pl.when(j == 0)
    def _():
        acc_ref[...] = jnp.zeros_like(acc_ref)

    acc_ref[...] += jnp.dot(adj_ref[...], u_ref[pl.ds(j * BN, BN), :],
                            preferred_element_type=jnp.float32)
    struct_ref[...] = jax.lax.dot_general(
        s1_ref[pl.ds(i * BM, BM), :], s1_ref[pl.ds(j * BN, BN), :],
        (((1,), (1,)), ((), ())), preferred_element_type=jnp.float32)

    @pl.when(j == pl.num_programs(1) - 1)
    def _():
        feat_ref[...] = (jnp.maximum(acc_ref[...], 0.0) * sc_ref[...]
                         + b_ref[...])


def kernel(y_features, adj, W_fd1, W_fd2, W_sd1, g1, b1, g2, b2, g3, b3):
    inv = 1.0 / jnp.sqrt(jnp.float32(1.0 + 1e-5))
    w_cat = jnp.concatenate([W_fd1, W_sd1], axis=1)
    scale_cat = (jnp.concatenate([g1, g3]) * inv).reshape(1, 2 * H1)
    beta_cat = jnp.concatenate([b1, b3]).reshape(1, 2 * H1)
    sc2 = (g2 * inv).reshape(1, D_IN)
    b2r = b2.reshape(1, D_IN)

    h, s1 = pl.pallas_call(
        _pass1_kernel,
        grid=(N // BM1,),
        in_specs=[
            pl.BlockSpec((BM1, N), lambda i: (i, 0)),
            pl.BlockSpec((N, H2), lambda i: (0, 0)),
            pl.BlockSpec((H2, 2 * H1), lambda i: (0, 0)),
            pl.BlockSpec((1, 2 * H1), lambda i: (0, 0)),
            pl.BlockSpec((1, 2 * H1), lambda i: (0, 0)),
        ],
        out_specs=[
            pl.BlockSpec((BM1, H1), lambda i: (i, 0)),
            pl.BlockSpec((BM1, H1), lambda i: (i, 0)),
        ],
        out_shape=[
            jax.ShapeDtypeStruct((N, H1), jnp.float32),
            jax.ShapeDtypeStruct((N, H1), jnp.float32),
        ],
        scratch_shapes=[pltpu.VMEM((N, 2 * H1), jnp.float32)],
    )(adj, y_features, w_cat, scale_cat, beta_cat)

    feat, struct = pl.pallas_call(
        _pass2_kernel,
        grid=(N // BM, N // BN),
        in_specs=[
            pl.BlockSpec((BM, BN), lambda i, j: (i, j)),
            pl.BlockSpec((N, H1), lambda i, j: (0, 0)),
            pl.BlockSpec((N, H1), lambda i, j: (0, 0)),
            pl.BlockSpec((H1, D_IN), lambda i, j: (0, 0)),
            pl.BlockSpec((1, D_IN), lambda i, j: (0, 0)),
            pl.BlockSpec((1, D_IN), lambda i, j: (0, 0)),
        ],
        out_specs=[
            pl.BlockSpec((BM, D_IN), lambda i, j: (i, 0)),
            pl.BlockSpec((BM, BN), lambda i, j: (i, j)),
        ],
        out_shape=[
            jax.ShapeDtypeStruct((N, D_IN), jnp.float32),
            jax.ShapeDtypeStruct((N, N), jnp.float32),
        ],
        scratch_shapes=[
            pltpu.VMEM((N, D_IN), jnp.float32),
            pltpu.VMEM((BM, D_IN), jnp.float32),
        ],
    )(adj, h, s1, W_fd2, sc2, b2r)

    return (feat, struct)


# speedup vs baseline: 1.6280x; 1.0332x over previous
import jax
import jax.numpy as jnp
from jax.experimental import pallas as pl
import jax.experimental.pallas.tpu as pltpu

N = 4096
H1 = 128
H2 = 64
D_IN = 256

BM = 1024
BN = 1024
NI = N // BM
NJ = N // BN
C = 6


def _fused_kernel(adj_ref, y_ref, w1_ref, w2_ref, scale_ref, beta_ref,
                  sc2_ref, b2_ref, feat_ref, struct_ref,
                  t_ref, u_ref, s1_ref, acc_ref, cache_ref):
    p = pl.program_id(0)
    i = pl.program_id(1)
    j = pl.program_id(2)
    lin = i * NJ + j

    @pl.when(jnp.logical_and(p == 0, lin == 0))
    def _():
        t_ref[...] = jnp.dot(y_ref[...], w1_ref[...],
                             preferred_element_type=jnp.float32)

    @pl.when(j == 0)
    def _():
        acc_ref[...] = jnp.zeros_like(acc_ref)

    @pl.when(p == 0)
    def _():
        acc_ref[...] += jnp.dot(adj_ref[...], t_ref[pl.ds(j * BN, BN), :],
                                preferred_element_type=jnp.float32)

        @pl.when(lin < C)
        def _():
            cache_ref[pl.ds(lin * BM, BM), :] = adj_ref[...]

        @pl.when(j == NJ - 1)
        def _():
            hs = (jnp.maximum(acc_ref[...], 0.0) * scale_ref[...]
                  + beta_ref[...])
            s1_ref[pl.ds(i * BM, BM), :] = hs[:, H1:]
            u_ref[pl.ds(i * BM, BM), :] = jnp.dot(
                hs[:, :H1], w2_ref[...], preferred_element_type=jnp.float32)

    @pl.when(p == 1)
    def _():
        u_j = u_ref[pl.ds(j * BN, BN), :]

        @pl.when(lin < C)
        def _():
            acc_ref[...] += jnp.dot(cache_ref[pl.ds(lin * BM, BM), :], u_j,
                                    preferred_element_type=jnp.float32)

        @pl.when(lin >= C)
        def _():
            acc_ref[...] += jnp.dot(adj_ref[...], u_j,
                                    preferred_element_type=jnp.float32)

        struct_ref[...] = jax.lax.dot_general(
            s1_ref[pl.ds(i * BM, BM), :], s1_ref[pl.ds(j * BN, BN), :],
            (((1,), (1,)), ((), ())), preferred_element_type=jnp.float32)

        @pl.when(j == NJ - 1)
        def _():
            feat_ref[...] = (jnp.maximum(acc_ref[...], 0.0) * sc2_ref[...]
                             + b2_ref[...])


def _adj_index_map(p, i, j):
    lin = i * NJ + j
    cached = jnp.logical_and(p == 1, lin < C)
    return (jnp.where(cached, C // NJ, i), jnp.where(cached, C % NJ, j))


def kernel(y_features, adj, W_fd1, W_fd2, W_sd1, g1, b1, g2, b2, g3, b3):
    inv = 1.0 / jnp.sqrt(jnp.float32(1.0 + 1e-5))
    w_cat = jnp.concatenate([W_fd1, W_sd1], axis=1)
    scale_cat = (jnp.concatenate([g1, g3]) * inv).reshape(1, 2 * H1)
    beta_cat = jnp.concatenate([b1, b3]).reshape(1, 2 * H1)
    sc2 = (g2 * inv).reshape(1, D_IN)
    b2r = b2.reshape(1, D_IN)

    feat, struct = pl.pallas_call(
        _fused_kernel,
        grid=(2, NI, NJ),
        in_specs=[
            pl.BlockSpec((BM, BN), _adj_index_map),
            pl.BlockSpec((N, H2), lambda p, i, j: (0, 0)),
            pl.BlockSpec((H2, 2 * H1), lambda p, i, j: (0, 0)),
            pl.BlockSpec((H1, D_IN), lambda p, i, j: (0, 0)),
            pl.BlockSpec((1, 2 * H1), lambda p, i, j: (0, 0)),
            pl.BlockSpec((1, 2 * H1), lambda p, i, j: (0, 0)),
            pl.BlockSpec((1, D_IN), lambda p, i, j: (0, 0)),
            pl.BlockSpec((1, D_IN), lambda p, i, j: (0, 0)),
        ],
        out_specs=[
            pl.BlockSpec((BM, D_IN),
                         lambda p, i, j: (jnp.where(p == 1, i, 0), 0)),
            pl.BlockSpec((BM, BN),
                         lambda p, i, j: (jnp.where(p == 1, i, 0),
                                          jnp.where(p == 1, j, 0))),
        ],
        out_shape=[
            jax.ShapeDtypeStruct((N, D_IN), jnp.float32),
            jax.ShapeDtypeStruct((N, N), jnp.float32),
        ],
        scratch_shapes=[
            pltpu.VMEM((N, D_IN), jnp.float32),
            pltpu.VMEM((N, D_IN), jnp.float32),
            pltpu.VMEM((N, H1), jnp.float32),
            pltpu.VMEM((BM, D_IN), jnp.float32),
            pltpu.VMEM((C * BM, BN), jnp.float32),
        ],
    )(adj, y_features, w_cat, W_fd2, scale_cat, beta_cat, sc2, b2r)

    return (feat, struct)


# speedup vs baseline: 1.6578x; 1.0183x over previous
import jax
import jax.numpy as jnp
from jax.experimental import pallas as pl
import jax.experimental.pallas.tpu as pltpu

N = 4096
H1 = 128
H2 = 64
D_IN = 256

BM = 1024
BN = 1024
NI = N // BM
NJ = N // BN
C = 7


def _fused_kernel(adj_ref, y_ref, w1_ref, w2_ref, scale_ref, beta_ref,
                  sc2_ref, b2_ref, feat_ref, struct_ref,
                  t_ref, u_ref, s1_ref, acc_ref, cache_ref):
    p = pl.program_id(0)
    i = pl.program_id(1)
    j = pl.program_id(2)
    lin = i * NJ + j

    @pl.when(jnp.logical_and(p == 0, lin == 0))
    def _():
        t_ref[...] = jnp.dot(y_ref[...], w1_ref[...],
                             preferred_element_type=jnp.float32)

    @pl.when(j == 0)
    def _():
        acc_ref[...] = jnp.zeros_like(acc_ref)

    @pl.when(p == 0)
    def _():
        acc_ref[...] += jnp.dot(adj_ref[...], t_ref[pl.ds(j * BN, BN), :],
                                preferred_element_type=jnp.float32)

        @pl.when(lin < C)
        def _():
            cache_ref[pl.ds(lin * BM, BM), :] = adj_ref[...]

        @pl.when(j == NJ - 1)
        def _():
            hs = (jnp.maximum(acc_ref[...], 0.0) * scale_ref[...]
                  + beta_ref[...])
            s1_ref[pl.ds(i * BM, BM), :] = hs[:, H1:].astype(jnp.bfloat16)
            u_ref[pl.ds(i * BM, BM), :] = jnp.dot(
                hs[:, :H1], w2_ref[...],
                preferred_element_type=jnp.float32).astype(jnp.bfloat16)

    @pl.when(p == 1)
    def _():
        u_j = u_ref[pl.ds(j * BN, BN), :].astype(jnp.float32)

        @pl.when(lin < C)
        def _():
            acc_ref[...] += jnp.dot(cache_ref[pl.ds(lin * BM, BM), :], u_j,
                                    preferred_element_type=jnp.float32)

        @pl.when(lin >= C)
        def _():
            acc_ref[...] += jnp.dot(adj_ref[...], u_j,
                                    preferred_element_type=jnp.float32)

        struct_ref[...] = jax.lax.dot_general(
            s1_ref[pl.ds(i * BM, BM), :], s1_ref[pl.ds(j * BN, BN), :],
            (((1,), (1,)), ((), ())), preferred_element_type=jnp.float32)

        @pl.when(j == NJ - 1)
        def _():
            feat_ref[...] = (jnp.maximum(acc_ref[...], 0.0) * sc2_ref[...]
                             + b2_ref[...])


def _adj_index_map(p, i, j):
    lin = i * NJ + j
    cached = jnp.logical_and(p == 1, lin < C)
    return (jnp.where(cached, C // NJ, i), jnp.where(cached, C % NJ, j))


def kernel(y_features, adj, W_fd1, W_fd2, W_sd1, g1, b1, g2, b2, g3, b3):
    inv = 1.0 / jnp.sqrt(jnp.float32(1.0 + 1e-5))
    w_cat = jnp.concatenate([W_fd1, W_sd1], axis=1)
    scale_cat = (jnp.concatenate([g1, g3]) * inv).reshape(1, 2 * H1)
    beta_cat = jnp.concatenate([b1, b3]).reshape(1, 2 * H1)
    sc2 = (g2 * inv).reshape(1, D_IN)
    b2r = b2.reshape(1, D_IN)

    feat, struct = pl.pallas_call(
        _fused_kernel,
        grid=(2, NI, NJ),
        in_specs=[
            pl.BlockSpec((BM, BN), _adj_index_map),
            pl.BlockSpec((N, H2), lambda p, i, j: (0, 0)),
            pl.BlockSpec((H2, 2 * H1), lambda p, i, j: (0, 0)),
            pl.BlockSpec((H1, D_IN), lambda p, i, j: (0, 0)),
            pl.BlockSpec((1, 2 * H1), lambda p, i, j: (0, 0)),
            pl.BlockSpec((1, 2 * H1), lambda p, i, j: (0, 0)),
            pl.BlockSpec((1, D_IN), lambda p, i, j: (0, 0)),
            pl.BlockSpec((1, D_IN), lambda p, i, j: (0, 0)),
        ],
        out_specs=[
            pl.BlockSpec((BM, D_IN),
                         lambda p, i, j: (jnp.where(p == 1, i, 0), 0)),
            pl.BlockSpec((BM, BN),
                         lambda p, i, j: (jnp.where(p == 1, i, 0),
                                          jnp.where(p == 1, j, 0))),
        ],
        out_shape=[
            jax.ShapeDtypeStruct((N, D_IN), jnp.float32),
            jax.ShapeDtypeStruct((N, N), jnp.float32),
        ],
        scratch_shapes=[
            pltpu.VMEM((N, D_IN), jnp.float32),
            pltpu.VMEM((N, D_IN), jnp.bfloat16),
            pltpu.VMEM((N, H1), jnp.bfloat16),
            pltpu.VMEM((BM, D_IN), jnp.float32),
            pltpu.VMEM((C * BM, BN), jnp.float32),
        ],
    )(adj, y_features, w_cat, W_fd2, scale_cat, beta_cat, sc2, b2r)

    return (feat, struct)
